# baseline (speedup 1.0000x reference)
import functools

import jax
import jax.numpy as jnp
from jax import lax
from jax.experimental import pallas as pl
from jax.experimental.pallas import tpu as pltpu
from jax.experimental.pallas import tpu_sc as plsc

_NW = 32
_CH = 31232
_NSEG = 4
_SEG = _CH // _NSEG
_LN2 = 0.6931471805599453
_SQRT2 = 1.4142135623730951


def _threefry2x32(x0, x1):
    ks0 = jnp.uint32(0)
    ks1 = jnp.uint32(1)
    ks2 = jnp.uint32(0x1BD11BDA) ^ ks0 ^ ks1
    ks = (ks0, ks1, ks2)
    rotations = ((13, 15, 26, 6), (17, 29, 16, 24))
    x0 = x0 + ks0
    x1 = x1 + ks1
    for i in range(5):
        for r in rotations[i % 2]:
            x0 = x0 + x1
            x1 = (x1 << r) | (x1 >> (32 - r))
            x1 = x1 ^ x0
        x0 = x0 + ks[(i + 1) % 3]
        x1 = x1 + ks[(i + 2) % 3] + jnp.uint32(i + 1)
    return x0, x1


def _bits_to_unit_float(bits):
    fb = (bits >> 9) | jnp.uint32(0x3F800000)
    return jax.lax.bitcast_convert_type(fb, jnp.float32) - 1.0


def _sample_row_vectors(p0, p1, p2, p3, u):
    c1 = p0 + p1
    c2 = c1 + p2
    c3 = c2 + p3
    g0 = u > p0
    g1 = u > c1
    g2 = u > c2
    g3 = u > c3
    one = jnp.ones_like(p0, dtype=jnp.int32)
    zero = jnp.zeros_like(one)
    ch = (jnp.where(g0, one, zero) + jnp.where(g1, one, zero)
          + jnp.where(g2, one, zero) + jnp.where(g3, one, zero))
    rp = jnp.where(g2, p3, jnp.where(g1, p2, jnp.where(g0, p1, p0)))
    return ch, rp


def _log_poly(rp):
    br = jax.lax.bitcast_convert_type(rp, jnp.uint32)
    ef = (((br >> 23) & jnp.uint32(0xFF)).astype(jnp.int32)
          - 127).astype(jnp.float32)
    m = jax.lax.bitcast_convert_type(
        (br & jnp.uint32(0x7FFFFF)) | jnp.uint32(0x3F800000), jnp.float32)
    big = m > _SQRT2
    m = jnp.where(big, m * 0.5, m)
    ef = jnp.where(big, ef + 1.0, ef)
    t = (m - 1.0) / (m + 1.0)
    z = t * t
    pol = t * (2.0 + z * (0.66666667 + z * (0.4 + z * 0.28571429)))
    return ef * _LN2 + pol


def _make_sc_kernel(NSC):
    mesh = plsc.VectorSubcoreMesh(core_axis_name="c", subcore_axis_name="s")

    @functools.partial(
        pl.kernel,
        out_type=[
            jax.ShapeDtypeStruct((NSC,), jnp.int32),
            jax.ShapeDtypeStruct((_NW, 16), jnp.float32),
        ],
        mesh=mesh,
        scratch_types=[
            pltpu.VMEM((4, _SEG), jnp.float32),
            pltpu.VMEM((_SEG,), jnp.float32),
            pltpu.VMEM((_SEG,), jnp.int32),
            pltpu.VMEM((16,), jnp.float32),
        ],
    )
    def sc_kernel(pt_hbm, q_hbm, ch_hbm, lp_hbm, p_v, q_v, c_v, a_v):
        wid = lax.axis_index("s") * 2 + lax.axis_index("c")
        wbase = wid * _CH
        lane = lax.iota(jnp.int32, 16).astype(jnp.uint32)

        acc = jnp.zeros((16,), jnp.float32)
        for s in range(_NSEG):
            base = pl.multiple_of(wbase + s * _SEG, 128)
            pltpu.sync_copy(pt_hbm.at[:, pl.ds(base, _SEG)], p_v)
            pltpu.sync_copy(q_hbm.at[pl.ds(base, _SEG)], q_v)

            def body(j, acc, base=base):
                for k in range(4):
                    sl = pl.ds(j * 64 + k * 16, 16)
                    ctr = (jnp.uint32(base)
                           + (j * 64 + k * 16).astype(jnp.uint32) + lane)
                    b0, b1 = _threefry2x32(jnp.zeros((16,), jnp.uint32), ctr)
                    u = _bits_to_unit_float(b0 ^ b1)
                    ch, rp = _sample_row_vectors(
                        p_v[0, sl], p_v[1, sl], p_v[2, sl], p_v[3, sl], u)
                    c_v[sl] = ch
                    acc = acc + q_v[sl] * _log_poly(rp)
                return acc

            acc = lax.fori_loop(0, _SEG // 64, body, acc)
            pltpu.sync_copy(c_v, ch_hbm.at[pl.ds(base, _SEG)])

        a_v[...] = acc
        pltpu.sync_copy(a_v, lp_hbm.at[wid])

    return sc_kernel


_NSC = _NW * _CH
_SC_KERNEL = _make_sc_kernel(_NSC)


def kernel(pmfs, q_values):
    B, A = pmfs.shape
    assert A == 4
    pt = pmfs.T

    ch_sc, lp_sc = _SC_KERNEL(pt, q_values)

    tail = B - _NSC
    pm_t = lax.slice(pmfs, (_NSC, 0), (B, 4))
    ctr = jnp.uint32(_NSC) + lax.iota(jnp.uint32, tail)
    b0, b1 = _threefry2x32(jnp.zeros_like(ctr), ctr)
    u_t = _bits_to_unit_float(b0 ^ b1)
    ch_t, rp_t = _sample_row_vectors(
        pm_t[:, 0], pm_t[:, 1], pm_t[:, 2], pm_t[:, 3], u_t)
    tail_sum = jnp.sum(lax.slice(q_values, (_NSC,), (B,)) * jnp.log(rp_t))

    loss = -(jnp.sum(lp_sc) + tail_sum) / B
    return (loss, jnp.concatenate([ch_sc, ch_t]))


# speedup vs baseline: 1.0717x; 1.0717x over previous
import functools

import jax
import jax.numpy as jnp
from jax import lax
from jax.experimental import pallas as pl
from jax.experimental.pallas import tpu as pltpu
from jax.experimental.pallas import tpu_sc as plsc

_NW = 32
_CH = 31232
_NSEG = 2
_SEG = _CH // _NSEG
_LN2 = 0.6931471805599453
_SQRT2 = 1.4142135623730951


def _threefry2x32(x0, x1):
    ks0 = jnp.uint32(0)
    ks1 = jnp.uint32(1)
    ks2 = jnp.uint32(0x1BD11BDA) ^ ks0 ^ ks1
    ks = (ks0, ks1, ks2)
    rotations = ((13, 15, 26, 6), (17, 29, 16, 24))
    x0 = x0 + ks0
    x1 = x1 + ks1
    for i in range(5):
        for r in rotations[i % 2]:
            x0 = x0 + x1
            x1 = (x1 << r) | (x1 >> (32 - r))
            x1 = x1 ^ x0
        x0 = x0 + ks[(i + 1) % 3]
        x1 = x1 + ks[(i + 2) % 3] + jnp.uint32(i + 1)
    return x0, x1


def _bits_to_unit_float(bits):
    fb = (bits >> 9) | jnp.uint32(0x3F800000)
    return jax.lax.bitcast_convert_type(fb, jnp.float32) - 1.0


def _sample_row_vectors(p0, p1, p2, p3, u):
    c1 = p0 + p1
    c2 = c1 + p2
    c3 = c2 + p3
    g0 = u > p0
    g1 = u > c1
    g2 = u > c2
    g3 = u > c3
    one = jnp.ones_like(p0, dtype=jnp.int32)
    zero = jnp.zeros_like(one)
    ch = (jnp.where(g0, one, zero) + jnp.where(g1, one, zero)
          + jnp.where(g2, one, zero) + jnp.where(g3, one, zero))
    rp = jnp.where(g2, p3, jnp.where(g1, p2, jnp.where(g0, p1, p0)))
    return ch, rp


def _log_poly(rp):
    br = jax.lax.bitcast_convert_type(rp, jnp.uint32)
    ef = (((br >> 23) & jnp.uint32(0xFF)).astype(jnp.int32)
          - 127).astype(jnp.float32)
    m = jax.lax.bitcast_convert_type(
        (br & jnp.uint32(0x7FFFFF)) | jnp.uint32(0x3F800000), jnp.float32)
    big = m > _SQRT2
    m = jnp.where(big, m * 0.5, m)
    ef = jnp.where(big, ef + 1.0, ef)
    t = (m - 1.0) / (m + 1.0)
    z = t * t
    pol = t * (2.0 + z * (0.66666667 + z * (0.4 + z * 0.28571429)))
    return ef * _LN2 + pol


def _make_sc_kernel(NSC):
    mesh = plsc.VectorSubcoreMesh(core_axis_name="c", subcore_axis_name="s")

    @functools.partial(
        pl.kernel,
        out_type=[
            jax.ShapeDtypeStruct((NSC,), jnp.int32),
            jax.ShapeDtypeStruct((_NW, 16), jnp.float32),
        ],
        mesh=mesh,
        scratch_types=[
            pltpu.VMEM((4, _SEG), jnp.float32),
            pltpu.VMEM((_SEG,), jnp.float32),
            pltpu.VMEM((_SEG,), jnp.int32),
            pltpu.VMEM((16,), jnp.float32),
        ],
    )
    def sc_kernel(pt_hbm, q_hbm, ch_hbm, lp_hbm, p_v, q_v, c_v, a_v):
        wid = lax.axis_index("s") * 2 + lax.axis_index("c")
        wbase = wid * _CH
        lane = lax.iota(jnp.int32, 16).astype(jnp.uint32)

        acc = jnp.zeros((16,), jnp.float32)
        for s in range(_NSEG):
            base = pl.multiple_of(wbase + s * _SEG, 128)
            pltpu.sync_copy(pt_hbm.at[:, pl.ds(base, _SEG)], p_v)
            pltpu.sync_copy(q_hbm.at[pl.ds(base, _SEG)], q_v)

            def body(j, acc, base=base):
                sl = pl.ds(j * 16, 16)
                ctr = (jnp.uint32(base) + (j * 16).astype(jnp.uint32)
                       + lane)
                b0, b1 = _threefry2x32(jnp.zeros((16,), jnp.uint32), ctr)
                u = _bits_to_unit_float(b0 ^ b1)
                ch, rp = _sample_row_vectors(
                    p_v[0, sl], p_v[1, sl], p_v[2, sl], p_v[3, sl], u)
                c_v[sl] = ch
                return acc + q_v[sl] * _log_poly(rp)

            acc = lax.fori_loop(0, _SEG // 16, body, acc)
            pltpu.sync_copy(c_v, ch_hbm.at[pl.ds(base, _SEG)])

        a_v[...] = acc
        pltpu.sync_copy(a_v, lp_hbm.at[wid])

    return sc_kernel


_NSC = _NW * _CH
_SC_KERNEL = _make_sc_kernel(_NSC)


def kernel(pmfs, q_values):
    B, A = pmfs.shape
    assert A == 4
    pt = pmfs.T

    ch_sc, lp_sc = _SC_KERNEL(pt, q_values)

    tail = B - _NSC
    pm_t = lax.slice(pmfs, (_NSC, 0), (B, 4))
    ctr = jnp.uint32(_NSC) + lax.iota(jnp.uint32, tail)
    b0, b1 = _threefry2x32(jnp.zeros_like(ctr), ctr)
    u_t = _bits_to_unit_float(b0 ^ b1)
    ch_t, rp_t = _sample_row_vectors(
        pm_t[:, 0], pm_t[:, 1], pm_t[:, 2], pm_t[:, 3], u_t)
    tail_sum = jnp.sum(lax.slice(q_values, (_NSC,), (B,)) * jnp.log(rp_t))

    loss = -(jnp.sum(lp_sc) + tail_sum) / B
    return (loss, jnp.concatenate([ch_sc, ch_t]))
